# Initial kernel scaffold; baseline (speedup 1.0000x reference)
import jax
import jax.numpy as jnp
from jax.experimental import pallas as pl


def kernel(keys, block_tables, cu_seqlens, pooling_heads_idx, num_retrieval_kv_heads):
    raise NotImplementedError("write your pallas kernel here")


# speedup vs baseline: 28.4743x; 28.4743x over previous
import functools

import jax
import jax.numpy as jnp
from jax import lax
from jax.experimental import pallas as pl
from jax.experimental.pallas import tpu as pltpu
from jax.experimental.pallas import tpu_sc as plsc

TOKENS_PER_BLOCK = 64
TOKENS_PER_SUB_CHUNK = 16
SUBS_PER_BLOCK = TOKENS_PER_BLOCK // TOKENS_PER_SUB_CHUNK
NUM_PAGES = 512

_CHUNK = 2048


def _pool_body(idx_ref, x_ref, o_ref):
    del idx_ref
    x = x_ref[...]
    xr = x.reshape(_CHUNK // TOKENS_PER_SUB_CHUNK, TOKENS_PER_SUB_CHUNK, 128)
    o_ref[0] = jnp.min(xr, axis=1)
    o_ref[1] = jnp.max(xr, axis=1)


def _pool(keys2, heads_idx, T, P, D):
    n_sub = T // TOKENS_PER_SUB_CHUNK
    spec = pltpu.PrefetchScalarGridSpec(
        num_scalar_prefetch=1,
        grid=(T // _CHUNK, P),
        in_specs=[pl.BlockSpec((_CHUNK, D), lambda i, p, idx: (i, idx[p]))],
        out_specs=pl.BlockSpec(
            (2, _CHUNK // TOKENS_PER_SUB_CHUNK, D), lambda i, p, idx: (0, i, p)
        ),
    )
    return pl.pallas_call(
        _pool_body,
        grid_spec=spec,
        out_shape=jax.ShapeDtypeStruct((2, n_sub, P * D), jnp.float32),
    )(heads_idx, keys2)


def _make_sc_scatter(n_blocks, row_f32, n_seq, max_blocks_per_seq, out_rows):
    mesh = plsc.VectorSubcoreMesh(core_axis_name="c", subcore_axis_name="s")
    blocks_per_sub = n_blocks // 16
    zrows = 16
    half = out_rows // 2

    @functools.partial(
        pl.kernel,
        mesh=mesh,
        out_type=jax.ShapeDtypeStruct((out_rows, row_f32), jnp.float32),
        scratch_types=[
            pltpu.VMEM((16,), jnp.int32),
            pltpu.VMEM((NUM_PAGES,), jnp.int32),
            pltpu.VMEM((16,), jnp.int32),
            pltpu.VMEM((16, row_f32), jnp.float32),
            pltpu.VMEM((zrows, row_f32), jnp.float32),
            pltpu.SemaphoreType.DMA,
        ],
        compiler_params=pltpu.CompilerParams(needs_layout_passes=False),
    )
    def sc_scatter(pooled_hbm, bt_hbm, cu_hbm, zeros_hbm, out_hbm,
                   cu_v, bt_v, idx_v, stage_v, zero_v, sem):
        c = lax.axis_index("c")
        s = lax.axis_index("s")
        pltpu.sync_copy(zeros_hbm, zero_v)
        rows_per_sub = half // 16
        base = c * half + s * rows_per_sub
        for r in range(0, rows_per_sub, zrows):
            pltpu.sync_copy(zero_v, out_hbm.at[pl.ds(base + r, zrows)])
        plsc.subcore_barrier()
        pltpu.sync_copy(cu_hbm, cu_v)
        pltpu.sync_copy(bt_hbm, bt_v)
        b_vec = s * blocks_per_sub + lax.iota(jnp.int32, 16)
        t_vec = b_vec * TOKENS_PER_BLOCK
        seq = jnp.zeros((16,), jnp.int32)
        for j in range(1, n_seq + 1):
            cj = plsc.load_gather(cu_v, [jnp.full((16,), j, jnp.int32)])
            seq = seq + (cj <= t_vec).astype(jnp.int32)
        cu_s = plsc.load_gather(cu_v, [seq])
        flat = seq * max_blocks_per_seq + (t_vec - cu_s) // TOKENS_PER_BLOCK
        pages = plsc.load_gather(bt_v, [flat])
        idx_v[...] = pages + c * half
        pltpu.sync_copy(pooled_hbm.at[pl.ds(c * n_blocks + s * blocks_per_sub, 16)],
                        stage_v)
        pltpu.async_copy(stage_v, out_hbm.at[idx_v], sem).wait()

    return sc_scatter


def kernel(keys, block_tables, cu_seqlens, pooling_heads_idx,
           num_retrieval_kv_heads):
    del num_retrieval_kv_heads
    T, H, D = keys.shape
    P = pooling_heads_idx.shape[0]
    n_seq = cu_seqlens.shape[0] - 1
    n_blocks = T // TOKENS_PER_BLOCK
    row_f32 = SUBS_PER_BLOCK * P * D

    keys2 = keys.reshape(T, H * D)
    heads_idx = pooling_heads_idx.astype(jnp.int32)
    pooled = _pool(keys2, heads_idx, T, P, D)
    pooled_rows = pooled.reshape(2 * n_blocks, row_f32)

    bt_flat = block_tables.reshape(-1).astype(jnp.int32)
    cu_pad = jnp.full((16,), 0x3FFFFFFF, jnp.int32)
    cu_pad = cu_pad.at[: cu_seqlens.shape[0]].set(cu_seqlens.astype(jnp.int32))
    zeros = jnp.zeros((16, row_f32), jnp.float32)

    scatter = _make_sc_scatter(n_blocks, row_f32, n_seq, block_tables.shape[1],
                               2 * NUM_PAGES)
    out = scatter(pooled_rows, bt_flat, cu_pad, zeros)
    return out.reshape(2, NUM_PAGES * SUBS_PER_BLOCK, P, D)


# speedup vs baseline: 61.7181x; 2.1675x over previous
import functools

import jax
import jax.numpy as jnp
from jax import lax
from jax.experimental import pallas as pl
from jax.experimental.pallas import tpu as pltpu
from jax.experimental.pallas import tpu_sc as plsc

TOKENS_PER_BLOCK = 64
TOKENS_PER_SUB_CHUNK = 16
SUBS_PER_BLOCK = TOKENS_PER_BLOCK // TOKENS_PER_SUB_CHUNK
NUM_PAGES = 512

_CHUNK = 1024


def _pool_body(x_ref, o_ref):
    x = x_ref[...]
    n, h, d = x.shape
    xr = x.reshape(n // TOKENS_PER_SUB_CHUNK, TOKENS_PER_SUB_CHUNK, h, d)
    o_ref[0] = jnp.min(xr, axis=1)
    o_ref[1] = jnp.max(xr, axis=1)


def _pool(keys, T, H, D):
    n_sub = T // TOKENS_PER_SUB_CHUNK
    return pl.pallas_call(
        _pool_body,
        grid=(T // _CHUNK,),
        in_specs=[pl.BlockSpec((_CHUNK, H, D), lambda i: (i, 0, 0))],
        out_specs=pl.BlockSpec(
            (2, _CHUNK // TOKENS_PER_SUB_CHUNK, H, D), lambda i: (0, i, 0, 0)
        ),
        out_shape=jax.ShapeDtypeStruct((2, n_sub, H, D), jnp.float32),
    )(keys)


def _make_sc_scatter(n_blocks, n_heads, n_pool, n_seq, max_blocks_per_seq):
    mesh = plsc.VectorSubcoreMesh(core_axis_name="c", subcore_axis_name="s")
    blocks_per_sub = n_blocks // 16
    rows_per_block = SUBS_PER_BLOCK * n_pool
    half_src = n_blocks * SUBS_PER_BLOCK * n_heads
    half_dst = NUM_PAGES * SUBS_PER_BLOCK * n_pool
    out_rows = 2 * half_dst
    n_idx = blocks_per_sub * rows_per_block
    zrows = 128

    @functools.partial(
        pl.kernel,
        mesh=mesh,
        out_type=jax.ShapeDtypeStruct((out_rows, 128), jnp.float32),
        scratch_types=[
            pltpu.VMEM((16,), jnp.int32),
            pltpu.VMEM((NUM_PAGES,), jnp.int32),
            pltpu.VMEM((16,), jnp.int32),
            pltpu.VMEM((128,), jnp.int32),
            pltpu.VMEM((128,), jnp.int32),
            pltpu.VMEM((128,), jnp.int32),
            pltpu.VMEM((128,), jnp.int32),
            pltpu.VMEM((n_idx, 128), jnp.float32),
            pltpu.VMEM((zrows, 128), jnp.float32),
            pltpu.SemaphoreType.DMA,
        ],
        compiler_params=pltpu.CompilerParams(needs_layout_passes=False),
    )
    def sc_scatter(pooled_hbm, bt_hbm, cu_hbm, heads_hbm, zeros_hbm, out_hbm,
                   cu_v, bt_v, heads_v, idx_src_a, idx_src_b,
                   idx_dst_a, idx_dst_b, stage_v, zero_v, sem):
        c = lax.axis_index("c")
        s = lax.axis_index("s")
        pltpu.sync_copy(zeros_hbm, zero_v)
        rows_per_sub = half_dst // 16
        base = c * half_dst + s * rows_per_sub
        for r in range(0, rows_per_sub, zrows):
            pltpu.sync_copy(zero_v, out_hbm.at[pl.ds(base + r, zrows)])
        plsc.subcore_barrier()
        pltpu.sync_copy(cu_hbm, cu_v)
        pltpu.sync_copy(bt_hbm, bt_v)
        pltpu.sync_copy(heads_hbm, heads_v)
        iota = lax.iota(jnp.int32, 16)
        b_vec = s * blocks_per_sub + iota
        t_vec = b_vec * TOKENS_PER_BLOCK
        seq = jnp.zeros((16,), jnp.int32)
        for j in range(1, n_seq + 1):
            cj = plsc.load_gather(cu_v, [jnp.full((16,), j, jnp.int32)])
            seq = seq + (cj <= t_vec).astype(jnp.int32)
        cu_s = plsc.load_gather(cu_v, [seq])
        flat = seq * max_blocks_per_seq + (t_vec - cu_s) // TOKENS_PER_BLOCK
        pages = plsc.load_gather(bt_v, [flat])
        src_base = (c * half_src
                    + (s * blocks_per_sub + iota) * (SUBS_PER_BLOCK * n_heads))
        dst_base = c * half_dst + pages * rows_per_block
        for k in range(rows_per_block):
            sub, h_slot = k // n_pool, k % n_pool
            head_val = plsc.load_gather(
                heads_v, [jnp.full((16,), h_slot + 1, jnp.int32)])
            dst = dst_base + (sub * n_pool + h_slot)
            src = src_base + sub * n_heads + head_val
            dref = idx_dst_a if k < 8 else idx_dst_b
            sref = idx_src_a if k < 8 else idx_src_b
            dref[pl.ds((k & 7) * 16, 16)] = dst
            sref[pl.ds((k & 7) * 16, 16)] = src
        pltpu.async_copy(pooled_hbm.at[idx_src_a],
                         stage_v.at[pl.ds(0, 128)], sem).wait()
        pltpu.async_copy(pooled_hbm.at[idx_src_b],
                         stage_v.at[pl.ds(128, 128)], sem).wait()
        pltpu.async_copy(stage_v.at[pl.ds(0, 128)],
                         out_hbm.at[idx_dst_a], sem).wait()
        pltpu.async_copy(stage_v.at[pl.ds(128, 128)],
                         out_hbm.at[idx_dst_b], sem).wait()

    return sc_scatter


def kernel(keys, block_tables, cu_seqlens, pooling_heads_idx,
           num_retrieval_kv_heads):
    del num_retrieval_kv_heads
    T, H, D = keys.shape
    P = pooling_heads_idx.shape[0]
    n_seq = cu_seqlens.shape[0] - 1
    n_blocks = T // TOKENS_PER_BLOCK

    pooled = _pool(keys, T, H, D)
    pooled_rows = pooled.reshape(2 * (T // TOKENS_PER_SUB_CHUNK) * H, D)

    bt_flat = block_tables.reshape(-1).astype(jnp.int32)
    cu_pad = jnp.full((16,), 0x3FFFFFFF, jnp.int32)
    cu_pad = cu_pad.at[: cu_seqlens.shape[0]].set(cu_seqlens.astype(jnp.int32))
    heads_pad = jnp.zeros((16,), jnp.int32)
    heads_pad = heads_pad.at[1 : P + 1].set(pooling_heads_idx.astype(jnp.int32))
    zeros = jnp.zeros((128, D), jnp.float32)

    scatter = _make_sc_scatter(n_blocks, H, P, n_seq, block_tables.shape[1])
    out = scatter(pooled_rows, bt_flat, cu_pad, heads_pad, zeros)
    return out.reshape(2, NUM_PAGES * SUBS_PER_BLOCK, P, D)


# speedup vs baseline: 65.9825x; 1.0691x over previous
import functools

import jax
import jax.numpy as jnp
from jax import lax
from jax.experimental import pallas as pl
from jax.experimental.pallas import tpu as pltpu
from jax.experimental.pallas import tpu_sc as plsc

TOKENS_PER_BLOCK = 64
TOKENS_PER_SUB_CHUNK = 16
SUBS_PER_BLOCK = TOKENS_PER_BLOCK // TOKENS_PER_SUB_CHUNK
NUM_PAGES = 512

_CHUNK = 1024


def _pool_body(x_ref, o_ref):
    x = x_ref[...]
    n, h, d = x.shape
    xr = x.reshape(n // TOKENS_PER_SUB_CHUNK, TOKENS_PER_SUB_CHUNK, h, d)
    o_ref[0] = jnp.min(xr, axis=1)
    o_ref[1] = jnp.max(xr, axis=1)


def _pool(keys, T, H, D):
    n_sub = T // TOKENS_PER_SUB_CHUNK
    return pl.pallas_call(
        _pool_body,
        grid=(T // _CHUNK,),
        in_specs=[pl.BlockSpec((_CHUNK, H, D), lambda i: (i, 0, 0))],
        out_specs=pl.BlockSpec(
            (2, _CHUNK // TOKENS_PER_SUB_CHUNK, H, D), lambda i: (0, i, 0, 0)
        ),
        out_shape=jax.ShapeDtypeStruct((2, n_sub, H, D), jnp.float32),
    )(keys)


def _make_sc_scatter(n_blocks, n_heads, n_pool, n_seq, max_blocks_per_seq):
    mesh = plsc.VectorSubcoreMesh(core_axis_name="c", subcore_axis_name="s")
    blocks_per_sub = n_blocks // 16
    rows_per_block = SUBS_PER_BLOCK * n_pool
    half_src = n_blocks * SUBS_PER_BLOCK * n_heads
    half_dst = NUM_PAGES * SUBS_PER_BLOCK * n_pool
    out_rows = 2 * half_dst
    n_idx = blocks_per_sub * rows_per_block
    zrows = 128

    @functools.partial(
        pl.kernel,
        mesh=mesh,
        out_type=jax.ShapeDtypeStruct((out_rows, 128), jnp.float32),
        scratch_types=[
            pltpu.VMEM((16,), jnp.int32),
            pltpu.VMEM((NUM_PAGES,), jnp.int32),
            pltpu.VMEM((16,), jnp.int32),
            pltpu.VMEM((128,), jnp.int32),
            pltpu.VMEM((128,), jnp.int32),
            pltpu.VMEM((128,), jnp.int32),
            pltpu.VMEM((128,), jnp.int32),
            pltpu.VMEM((n_idx, 128), jnp.float32),
            pltpu.VMEM((zrows, 128), jnp.float32),
            pltpu.SemaphoreType.DMA,
        ],
        compiler_params=pltpu.CompilerParams(needs_layout_passes=False),
    )
    def sc_scatter(pooled_hbm, bt_hbm, cu_hbm, heads_hbm, zeros_hbm, out_hbm,
                   cu_v, bt_v, heads_v, idx_src_a, idx_src_b,
                   idx_dst_a, idx_dst_b, stage_v, zero_v, sem):
        c = lax.axis_index("c")
        s = lax.axis_index("s")
        ld = [pltpu.async_copy(cu_hbm, cu_v, sem),
              pltpu.async_copy(bt_hbm, bt_v, sem),
              pltpu.async_copy(heads_hbm, heads_v, sem),
              pltpu.async_copy(zeros_hbm, zero_v, sem)]
        for h in ld:
            h.wait()
        rows_per_sub = half_dst // 16
        base = c * half_dst + s * rows_per_sub
        zfill = [
            pltpu.async_copy(zero_v, out_hbm.at[pl.ds(base + r, zrows)], sem)
            for r in range(0, rows_per_sub, zrows)
        ]
        iota = lax.iota(jnp.int32, 16)
        b_vec = s * blocks_per_sub + iota
        t_vec = b_vec * TOKENS_PER_BLOCK
        seq = jnp.zeros((16,), jnp.int32)
        for j in range(1, n_seq + 1):
            cj = plsc.load_gather(cu_v, [jnp.full((16,), j, jnp.int32)])
            seq = seq + (cj <= t_vec).astype(jnp.int32)
        cu_s = plsc.load_gather(cu_v, [seq])
        flat = seq * max_blocks_per_seq + (t_vec - cu_s) // TOKENS_PER_BLOCK
        pages = plsc.load_gather(bt_v, [flat])
        src_base = (c * half_src
                    + (s * blocks_per_sub + iota) * (SUBS_PER_BLOCK * n_heads))
        dst_base = c * half_dst + pages * rows_per_block
        for k in range(rows_per_block):
            sub, h_slot = k // n_pool, k % n_pool
            head_val = plsc.load_gather(
                heads_v, [jnp.full((16,), h_slot + 1, jnp.int32)])
            dst = dst_base + (sub * n_pool + h_slot)
            src = src_base + sub * n_heads + head_val
            dref = idx_dst_a if k < 8 else idx_dst_b
            sref = idx_src_a if k < 8 else idx_src_b
            dref[pl.ds((k & 7) * 16, 16)] = dst
            sref[pl.ds((k & 7) * 16, 16)] = src
        g0 = pltpu.async_copy(pooled_hbm.at[idx_src_a],
                              stage_v.at[pl.ds(0, 128)], sem)
        g1 = pltpu.async_copy(pooled_hbm.at[idx_src_b],
                              stage_v.at[pl.ds(128, 128)], sem)
        for h in zfill:
            h.wait()
        g0.wait()
        g1.wait()
        plsc.subcore_barrier()
        s0 = pltpu.async_copy(stage_v.at[pl.ds(0, 128)],
                              out_hbm.at[idx_dst_a], sem)
        s1 = pltpu.async_copy(stage_v.at[pl.ds(128, 128)],
                              out_hbm.at[idx_dst_b], sem)
        s0.wait()
        s1.wait()

    return sc_scatter


def kernel(keys, block_tables, cu_seqlens, pooling_heads_idx,
           num_retrieval_kv_heads):
    del num_retrieval_kv_heads
    T, H, D = keys.shape
    P = pooling_heads_idx.shape[0]
    n_seq = cu_seqlens.shape[0] - 1
    n_blocks = T // TOKENS_PER_BLOCK

    pooled = _pool(keys, T, H, D)
    pooled_rows = pooled.reshape(2 * (T // TOKENS_PER_SUB_CHUNK) * H, D)

    bt_flat = block_tables.reshape(-1).astype(jnp.int32)
    cu_pad = jnp.full((16,), 0x3FFFFFFF, jnp.int32)
    cu_pad = cu_pad.at[: cu_seqlens.shape[0]].set(cu_seqlens.astype(jnp.int32))
    heads_pad = jnp.zeros((16,), jnp.int32)
    heads_pad = heads_pad.at[1 : P + 1].set(pooling_heads_idx.astype(jnp.int32))
    zeros = jnp.zeros((128, D), jnp.float32)

    scatter = _make_sc_scatter(n_blocks, H, P, n_seq, block_tables.shape[1])
    out = scatter(pooled_rows, bt_flat, cu_pad, heads_pad, zeros)
    return out.reshape(2, NUM_PAGES * SUBS_PER_BLOCK, P, D)
